# baseline (speedup 1.0000x reference)
import functools

import jax
import jax.numpy as jnp
from jax import lax
from jax.experimental import pallas as pl
from jax.experimental.pallas import tpu as pltpu
from jax.experimental.pallas import tpu_sc as plsc

_EMBD = 64
_BATCH = 16384
_NC = 2
_NS = 16
_NW = _NC * _NS
_BPW = _BATCH // _NW
_CHUNK = 128
_NCHUNK = _BPW // _CHUNK
_LANES = 16


def _sc_body(fo_hbm, co_hbm, tab_hbm, out_hbm,
             fidx, cidx, frows, crows, accv, fsem, csem):
    wid = lax.axis_index("s") * _NC + lax.axis_index("c")
    pltpu.sync_copy(fo_hbm.at[wid], fidx)
    pltpu.sync_copy(co_hbm.at[wid], cidx)
    acc = jnp.zeros((_LANES,), jnp.float32)
    for j in range(_NCHUNK):
        fcp = pltpu.async_copy(tab_hbm.at[fidx.at[j]], frows, fsem)
        ccp = pltpu.async_copy(tab_hbm.at[cidx.at[j]], crows, csem)
        fcp.wait()
        ccp.wait()

        def row(i, a):
            for c in range(_EMBD // _LANES):
                a = a + (frows[i, pl.ds(c * _LANES, _LANES)]
                         * crows[i, pl.ds(c * _LANES, _LANES)])
            return a

        acc = lax.fori_loop(0, _CHUNK, row, acc)
    accv[...] = acc
    pltpu.sync_copy(accv, out_hbm.at[pl.ds(wid * _LANES, _LANES)])


_sc_partials = functools.partial(
    pl.kernel,
    out_type=jax.ShapeDtypeStruct((_NW * _LANES,), jnp.float32),
    mesh=plsc.VectorSubcoreMesh(core_axis_name="c", subcore_axis_name="s"),
    scratch_types=[
        pltpu.VMEM((_NCHUNK, _CHUNK), jnp.int32),
        pltpu.VMEM((_NCHUNK, _CHUNK), jnp.int32),
        pltpu.VMEM((_CHUNK, _EMBD), jnp.float32),
        pltpu.VMEM((_CHUNK, _EMBD), jnp.float32),
        pltpu.VMEM((_LANES,), jnp.float32),
        pltpu.SemaphoreType.DMA,
        pltpu.SemaphoreType.DMA,
    ],
    compiler_params=pltpu.CompilerParams(use_tc_tiling_on_sc=False),
)(_sc_body)


def _finish_body(p_ref, o_ref):
    o_ref[...] = jax.nn.log_sigmoid(jnp.sum(p_ref[...])).reshape(1, 1)


_finish = pl.pallas_call(
    _finish_body,
    out_shape=jax.ShapeDtypeStruct((1, 1), jnp.float32),
)


def kernel(focus, context, embeddings):
    fo = focus.reshape(_NW, _NCHUNK, _CHUNK)
    co = context.reshape(_NW, _NCHUNK, _CHUNK)
    partials = _sc_partials(fo, co, embeddings)
    return _finish(partials.reshape(4, 128))
